# Initial kernel scaffold; baseline (speedup 1.0000x reference)
import functools

import jax
import jax.numpy as jnp
from jax import lax
from jax.experimental import pallas as pl
from jax.experimental.pallas import tpu as pltpu
from jax.experimental.pallas import tpu_sc as plsc

B = 64
T = 512
TEXT_VOCAB = 256
D_MODEL = 128
N_EMOTIONS = 8
MEL_DIM = 80
CH = 128


def _tables_body(text_ref, emo_ref, emo_tab_ref, text_tab_ref, w_ref, b_ref,
                 big_ref, idx_ref):
    tp = jnp.dot(text_tab_ref[...], w_ref[...],
                 preferred_element_type=jnp.float32) + b_ref[...]
    ep = jnp.dot(emo_tab_ref[...], w_ref[...],
                 preferred_element_type=jnp.float32)
    big_ref[...] = tp[None, :, :] + ep[:, None, :]
    idx_ref[...] = text_ref[...] + emo_ref[...] * TEXT_VOCAB


def _build_tables(text_tensor, emotion_label, emotion_table, text_table,
                  W_out, b_out):
    return pl.pallas_call(
        _tables_body,
        out_shape=(
            jax.ShapeDtypeStruct((N_EMOTIONS, TEXT_VOCAB, MEL_DIM), jnp.float32),
            jax.ShapeDtypeStruct((B, T), jnp.int32),
        ),
    )(text_tensor, emotion_label.reshape(B, 1), emotion_table, text_table,
      W_out, b_out.reshape(1, MEL_DIM))


def _sc_gather(idx, table):
    info = plsc.get_sparse_core_info()
    nc, ns = info.num_cores, info.num_subcores
    nw = nc * ns
    tok = B * T
    per_w = tok // nw
    nch = per_w // CH
    idx3 = idx.reshape(nw, nch, CH)
    mesh = plsc.VectorSubcoreMesh(core_axis_name="c", subcore_axis_name="s")

    @functools.partial(
        pl.kernel,
        out_type=jax.ShapeDtypeStruct((tok, MEL_DIM), jnp.float32),
        mesh=mesh,
        scratch_types=[
            pltpu.VMEM((nch, CH), jnp.int32),
            pltpu.VMEM((per_w, MEL_DIM), jnp.float32),
            pltpu.SemaphoreType.DMA,
        ],
    )
    def gather_kernel(idx_hbm, table_hbm, out_hbm, idx_v, rows_v, sem):
        wid = lax.axis_index("s") * nc + lax.axis_index("c")
        pltpu.sync_copy(idx_hbm.at[wid], idx_v)
        copies = [
            pltpu.async_copy(table_hbm.at[idx_v.at[j]],
                             rows_v.at[pl.ds(j * CH, CH)], sem)
            for j in range(nch)
        ]
        for c in copies:
            c.wait()
        pltpu.sync_copy(rows_v, out_hbm.at[pl.ds(wid * per_w, per_w)])

    return gather_kernel(idx3, table)


def kernel(text_tensor, emotion_label, emotion_table, text_table, W_out, b_out):
    big, idx = _build_tables(text_tensor, emotion_label, emotion_table,
                             text_table, W_out, b_out)
    out = _sc_gather(idx, big.reshape(N_EMOTIONS * TEXT_VOCAB, MEL_DIM))
    return out.reshape(B, T, MEL_DIM)


# speedup vs baseline: 1.8563x; 1.8563x over previous
import functools

import jax
import jax.numpy as jnp
from jax import lax
from jax.experimental import pallas as pl
from jax.experimental.pallas import tpu as pltpu
from jax.experimental.pallas import tpu_sc as plsc

B = 64
T = 512
TEXT_VOCAB = 256
D_MODEL = 128
N_EMOTIONS = 8
MEL_DIM = 80
CH = 128


def _tables_body(text_ref, emo_ref, emo_tab_ref, text_tab_ref, w_ref, b_ref,
                 big_ref, idx_ref):
    tp = jnp.dot(text_tab_ref[...], w_ref[...],
                 preferred_element_type=jnp.float32) + b_ref[...]
    ep = jnp.dot(emo_tab_ref[...], w_ref[...],
                 preferred_element_type=jnp.float32)
    big_ref[...] = tp[None, :, :] + ep[:, None, :]
    idx_ref[...] = text_ref[...] + emo_ref[...] * TEXT_VOCAB


def _build_tables(text_tensor, emotion_label, emotion_table, text_table,
                  W_out, b_out):
    return pl.pallas_call(
        _tables_body,
        out_shape=(
            jax.ShapeDtypeStruct((N_EMOTIONS, TEXT_VOCAB, MEL_DIM), jnp.float32),
            jax.ShapeDtypeStruct((B, T), jnp.int32),
        ),
    )(text_tensor, emotion_label.reshape(B, 1), emotion_table, text_table,
      W_out, b_out.reshape(1, MEL_DIM))


def _sc_gather(idx, table):
    info = plsc.get_sparse_core_info()
    nc, ns = info.num_cores, info.num_subcores
    nw = nc * ns
    tok = B * T
    per_w = tok // nw
    nch = per_w // CH
    idx3 = idx.reshape(nw, nch, CH)
    mesh = plsc.VectorSubcoreMesh(core_axis_name="c", subcore_axis_name="s")

    @functools.partial(
        pl.kernel,
        out_type=jax.ShapeDtypeStruct((tok, MEL_DIM), jnp.float32),
        mesh=mesh,
        scratch_types=[
            pltpu.VMEM((nch, CH), jnp.int32),
            pltpu.VMEM((per_w, MEL_DIM), jnp.float32),
            pltpu.SemaphoreType.DMA,
        ],
        compiler_params=pltpu.CompilerParams(use_tc_tiling_on_sc=False),
    )
    def gather_kernel(idx_hbm, table_hbm, out_hbm, idx_v, rows_v, sem):
        wid = lax.axis_index("s") * nc + lax.axis_index("c")
        pltpu.sync_copy(idx_hbm.at[wid], idx_v)
        copies = [
            pltpu.async_copy(table_hbm.at[idx_v.at[j]],
                             rows_v.at[pl.ds(j * CH, CH)], sem)
            for j in range(nch)
        ]
        for c in copies:
            c.wait()
        pltpu.sync_copy(rows_v, out_hbm.at[pl.ds(wid * per_w, per_w)])

    return gather_kernel(idx3, table)


def kernel(text_tensor, emotion_label, emotion_table, text_table, W_out, b_out):
    big, idx = _build_tables(text_tensor, emotion_label, emotion_table,
                             text_table, W_out, b_out)
    out = _sc_gather(idx, big.reshape(N_EMOTIONS * TEXT_VOCAB, MEL_DIM))
    return out.reshape(B, T, MEL_DIM)


# speedup vs baseline: 2.2435x; 1.2086x over previous
import functools

import jax
import jax.numpy as jnp
from jax import lax
from jax.experimental import pallas as pl
from jax.experimental.pallas import tpu as pltpu
from jax.experimental.pallas import tpu_sc as plsc

B = 64
T = 512
TEXT_VOCAB = 256
D_MODEL = 128
N_EMOTIONS = 8
MEL_DIM = 80
LANES = 128
CH = 128
NBUF = 4


def _tables_body(text_ref, emo_ref, emo_tab_ref, text_tab_ref, w_ref, b_ref,
                 big_ref, idx_ref):
    tp = jnp.dot(text_tab_ref[...], w_ref[...],
                 preferred_element_type=jnp.float32) + b_ref[...]
    ep = jnp.dot(emo_tab_ref[...], w_ref[...],
                 preferred_element_type=jnp.float32)
    big_ref[...] = tp[None, :, :] + ep[:, None, :]
    idx_ref[...] = text_ref[...] + emo_ref[...] * TEXT_VOCAB


def _build_tables(text_tensor, emotion_label, emotion_table, text_table,
                  W_out, b_out):
    w_pad = jnp.pad(W_out, ((0, 0), (0, LANES - MEL_DIM)))
    b_pad = jnp.pad(b_out, (0, LANES - MEL_DIM)).reshape(1, LANES)
    return pl.pallas_call(
        _tables_body,
        out_shape=(
            jax.ShapeDtypeStruct((N_EMOTIONS, TEXT_VOCAB, LANES), jnp.float32),
            jax.ShapeDtypeStruct((B, T), jnp.int32),
        ),
    )(text_tensor, emotion_label.reshape(B, 1), emotion_table, text_table,
      w_pad, b_pad)


def _sc_gather(idx, table):
    info = plsc.get_sparse_core_info()
    nc, ns = info.num_cores, info.num_subcores
    nw = nc * ns
    tok = B * T
    per_w = tok // nw
    nch = per_w // CH
    idx3 = idx.reshape(nw, nch, CH)
    mesh = plsc.VectorSubcoreMesh(core_axis_name="c", subcore_axis_name="s")

    @functools.partial(
        pl.kernel,
        out_type=jax.ShapeDtypeStruct((tok, LANES), jnp.float32),
        mesh=mesh,
        scratch_types=[
            pltpu.VMEM((nch, CH), jnp.int32),
            pltpu.VMEM((NBUF, CH, LANES), jnp.float32),
            pltpu.SemaphoreType.DMA,
            pltpu.SemaphoreType.DMA,
        ],
        compiler_params=pltpu.CompilerParams(use_tc_tiling_on_sc=False),
    )
    def gather_kernel(idx_hbm, table_hbm, out_hbm, idx_v, rows_v, gsem, osem):
        wid = lax.axis_index("s") * nc + lax.axis_index("c")
        base = wid * per_w
        pltpu.sync_copy(idx_hbm.at[wid], idx_v)
        for p in range(nch // NBUF):
            gathers = [
                pltpu.async_copy(table_hbm.at[idx_v.at[p * NBUF + k]],
                                 rows_v.at[k], gsem)
                for k in range(NBUF)
            ]
            for g in gathers:
                g.wait()
            stores = [
                pltpu.async_copy(
                    rows_v.at[k],
                    out_hbm.at[pl.ds(base + (p * NBUF + k) * CH, CH)], osem)
                for k in range(NBUF)
            ]
            for s in stores:
                s.wait()

    return gather_kernel(idx3, table)


def kernel(text_tensor, emotion_label, emotion_table, text_table, W_out, b_out):
    big, idx = _build_tables(text_tensor, emotion_label, emotion_table,
                             text_table, W_out, b_out)
    out = _sc_gather(idx, big.reshape(N_EMOTIONS * TEXT_VOCAB, LANES))
    return out.reshape(B, T, LANES)[..., :MEL_DIM]
